# Initial kernel scaffold; baseline (speedup 1.0000x reference)
import jax
import jax.numpy as jnp
from jax.experimental import pallas as pl


def kernel(x, y, eps):
    raise NotImplementedError("write your pallas kernel here")


# speedup vs baseline: 2.1844x; 2.1844x over previous
import functools

import jax
import jax.numpy as jnp
from jax import lax
from jax.experimental import pallas as pl
from jax.experimental.pallas import tpu as pltpu

N = 320000
D = 128
C = 1000
B = 512
NB = N // B


def _seg_kernel(y_ref, x_ref, eps_ref, out_ref, sum_ref, ss_ref, cnt_ref):
    i = pl.program_id(0)

    @pl.when(i == 0)
    def _init():
        sum_ref[...] = jnp.zeros_like(sum_ref)
        ss_ref[...] = jnp.zeros_like(ss_ref)
        cnt_ref[...] = jnp.zeros_like(cnt_ref)

    xb = x_ref[...]
    yb = y_ref[0, 0, :]
    cls = lax.broadcasted_iota(jnp.int32, (C, B), 0)
    onehot = (cls == yb[None, :]).astype(jnp.float32)
    xcat = jnp.concatenate([xb, xb * xb], axis=1)
    part = jnp.dot(onehot, xcat, preferred_element_type=jnp.float32,
                   precision=lax.Precision.HIGHEST)
    sum_ref[...] += part[:, :D]
    ss_ref[...] += part[:, D:]
    cnt_ref[...] += jnp.sum(onehot, axis=1, keepdims=True)

    @pl.when(i == NB - 1)
    def _finalize():
        cnt = cnt_ref[...]
        denom = jnp.maximum(cnt, 1.0)
        s = sum_ref[...]
        ss = ss_ref[...]
        m = s / denom
        sq = ss - 2.0 * m * s + cnt * m * m
        sq = jnp.maximum(sq, 0.0)
        stddev = jnp.sqrt(sq / denom)
        e = jnp.clip(eps_ref[...], -2.0, 2.0)
        out_ref[...] = m + e * stddev


@jax.jit
def kernel(x, y, eps):
    y3 = y.astype(jnp.int32).reshape(NB, 1, B)
    return pl.pallas_call(
        _seg_kernel,
        grid=(NB,),
        in_specs=[
            pl.BlockSpec((1, 1, B), lambda i: (i, 0, 0)),
            pl.BlockSpec((B, D), lambda i: (i, 0)),
            pl.BlockSpec((C, D), lambda i: (0, 0)),
        ],
        out_specs=pl.BlockSpec((C, D), lambda i: (0, 0)),
        out_shape=jax.ShapeDtypeStruct((C, D), jnp.float32),
        scratch_shapes=[
            pltpu.VMEM((C, D), jnp.float32),
            pltpu.VMEM((C, D), jnp.float32),
            pltpu.VMEM((C, 1), jnp.float32),
        ],
    )(y3, x, eps)


# speedup vs baseline: 5.7552x; 2.6346x over previous
import functools

import jax
import jax.numpy as jnp
from jax import lax
from jax.experimental import pallas as pl
from jax.experimental.pallas import tpu as pltpu

N = 320000
D = 128
C = 1000
B = 512
NB = N // B


def _seg_kernel(y_ref, x_ref, eps_ref, out_ref, sum_ref, ss_ref, cnt_ref):
    i = pl.program_id(0)

    @pl.when(i == 0)
    def _init():
        sum_ref[...] = jnp.zeros_like(sum_ref)
        ss_ref[...] = jnp.zeros_like(ss_ref)
        cnt_ref[...] = jnp.zeros_like(cnt_ref)

    xb = x_ref[...]
    yb = y_ref[0, 0, :]
    cls = lax.broadcasted_iota(jnp.int32, (C, B), 0)
    onehot = (cls == yb[None, :]).astype(jnp.float32)
    xcat = jnp.concatenate([xb, xb * xb], axis=1)
    part = jnp.dot(onehot, xcat, preferred_element_type=jnp.float32,
                   precision=lax.Precision.DEFAULT)
    sum_ref[...] += part[:, :D]
    ss_ref[...] += part[:, D:]
    cnt_ref[...] += jnp.sum(onehot, axis=1, keepdims=True)

    @pl.when(i == NB - 1)
    def _finalize():
        cnt = cnt_ref[...]
        denom = jnp.maximum(cnt, 1.0)
        s = sum_ref[...]
        ss = ss_ref[...]
        m = s / denom
        sq = ss - 2.0 * m * s + cnt * m * m
        sq = jnp.maximum(sq, 0.0)
        stddev = jnp.sqrt(sq / denom)
        e = jnp.clip(eps_ref[...], -2.0, 2.0)
        out_ref[...] = m + e * stddev


@jax.jit
def kernel(x, y, eps):
    y3 = y.astype(jnp.int32).reshape(NB, 1, B)
    return pl.pallas_call(
        _seg_kernel,
        grid=(NB,),
        in_specs=[
            pl.BlockSpec((1, 1, B), lambda i: (i, 0, 0)),
            pl.BlockSpec((B, D), lambda i: (i, 0)),
            pl.BlockSpec((C, D), lambda i: (0, 0)),
        ],
        out_specs=pl.BlockSpec((C, D), lambda i: (0, 0)),
        out_shape=jax.ShapeDtypeStruct((C, D), jnp.float32),
        scratch_shapes=[
            pltpu.VMEM((C, D), jnp.float32),
            pltpu.VMEM((C, D), jnp.float32),
            pltpu.VMEM((C, 1), jnp.float32),
        ],
    )(y3, x, eps)
